# Initial kernel scaffold; baseline (speedup 1.0000x reference)
import jax
import jax.numpy as jnp
from jax.experimental import pallas as pl


def kernel(x, edge_index, ptr, target_node_idx, ego_node_idx, W_self, W_nbr, fc_w, fc_b, codebook):
    raise NotImplementedError("write your pallas kernel here")


# speedup vs baseline: 4.6012x; 4.6012x over previous
import functools

import jax
import jax.numpy as jnp
from jax import lax
from jax.experimental import pallas as pl
from jax.experimental.pallas import tpu as pltpu
from jax.experimental.pallas import tpu_sc as plsc

N = 10000
E = 320000
D = 128
EMB = 256
HID = 512
VOCAB = 8192
B = 1000

NW = 32
K = 128
CH = 80
EWP = CH * K
XW = 144
NPAD = N + 112
G = 2048
GW = G // NW
DUMMY = G
ACC_ROWS = 2176
RPT = ACC_ROWS // 16
NBV = 8
BV = VOCAB // NBV


def _sc_mesh():
    return plsc.VectorSubcoreMesh(core_axis_name="c", subcore_axis_name="s")


_SC_PARAMS = pltpu.CompilerParams(use_tc_tiling_on_sc=False,
                                  needs_layout_passes=False)


@functools.partial(
    pl.kernel,
    mesh=_sc_mesh(),
    compiler_params=_SC_PARAMS,
    out_type=(
        jax.ShapeDtypeStruct((ACC_ROWS, XW), jnp.float32),
        jax.ShapeDtypeStruct((ACC_ROWS, XW), jnp.float32),
    ),
    scratch_types=[
        pltpu.VMEM((CH, K), jnp.int32),
        pltpu.VMEM((CH, K), jnp.int32),
        pltpu.VMEM((NPAD,), jnp.int32),
        pltpu.VMEM((K, XW), jnp.float32),
        pltpu.VMEM((K, XW), jnp.float32),
        pltpu.VMEM((K, XW), jnp.float32),
        pltpu.VMEM((K, XW), jnp.float32),
        pltpu.VMEM_SHARED((ACC_ROWS, XW), jnp.float32),
        pltpu.SemaphoreType.DMA,
        pltpu.SemaphoreType.DMA,
        pltpu.SemaphoreType.DMA,
        pltpu.SemaphoreType.DMA,
    ],
)
def _edge_agg(xext_h, srcp_h, dstp_h, slot_h, zero_h, out_a, out_b,
              src_v, dst_v, slot_v, r0, r1, r2, r3, acc_sh,
              sem0, sem1, sem2, sem3):
    cid = lax.axis_index("c")
    sid = lax.axis_index("s")
    wid = sid * 2 + cid
    pltpu.sync_copy(zero_h.at[pl.ds(sid * RPT, RPT)],
                    acc_sh.at[pl.ds(sid * RPT, RPT)])
    plsc.subcore_barrier()
    pltpu.sync_copy(srcp_h.at[wid], src_v)
    pltpu.sync_copy(dstp_h.at[wid], dst_v)
    pltpu.sync_copy(slot_h, slot_v)

    def translate(j, carry):
        for k in range(K // 16):
            d16 = dst_v[j, pl.ds(k * 16, 16)]
            dst_v[j, pl.ds(k * 16, 16)] = plsc.load_gather(slot_v, [d16])
        return carry

    lax.fori_loop(0, CH, translate, 0)

    bufs = (r0, r1, r2, r3)
    sems = (sem0, sem1, sem2, sem3)
    NB = len(bufs)

    def group(g, carry):
        base = g * NB
        copies = [
            pltpu.async_copy(xext_h.at[src_v.at[base + i]], bufs[i], sems[i])
            for i in range(NB)
        ]
        for i in range(NB):
            copies[i].wait()
            pltpu.sync_copy(bufs[i], acc_sh.at[dst_v.at[base + i]], add=True)
        return carry

    lax.fori_loop(0, CH // NB, group, 0)
    plsc.subcore_barrier()

    @pl.when(cid == 0)
    def _():
        pltpu.sync_copy(acc_sh.at[pl.ds(sid * RPT, RPT)],
                        out_a.at[pl.ds(sid * RPT, RPT)])

    @pl.when(cid == 1)
    def _():
        pltpu.sync_copy(acc_sh.at[pl.ds(sid * RPT, RPT)],
                        out_b.at[pl.ds(sid * RPT, RPT)])


@functools.partial(
    pl.kernel,
    mesh=_sc_mesh(),
    compiler_params=_SC_PARAMS,
    out_type=(
        jax.ShapeDtypeStruct((G, XW), jnp.float32),
        jax.ShapeDtypeStruct((G, XW), jnp.float32),
        jax.ShapeDtypeStruct((G, XW), jnp.float32),
    ),
    scratch_types=[
        pltpu.VMEM((GW,), jnp.int32),
        pltpu.VMEM((GW,), jnp.int32),
        pltpu.VMEM((GW, XW), jnp.float32),
        pltpu.VMEM((GW, XW), jnp.float32),
        pltpu.VMEM((GW, XW), jnp.float32),
        pltpu.SemaphoreType.DMA,
        pltpu.SemaphoreType.DMA,
        pltpu.SemaphoreType.DMA,
    ],
)
def _row_gather(xext_h, acc_a, acc_b, idx_h, canon_h, xg_h, ga_h, gb_h,
                idx_v, can_v, bx, ba, bb, sem0, sem1, sem2):
    cid = lax.axis_index("c")
    sid = lax.axis_index("s")
    wid = sid * 2 + cid
    pltpu.sync_copy(idx_h.at[wid], idx_v)
    pltpu.sync_copy(canon_h.at[wid], can_v)
    c1 = pltpu.async_copy(xext_h.at[idx_v], bx, sem0)
    c2 = pltpu.async_copy(acc_a.at[can_v], ba, sem1)
    c3 = pltpu.async_copy(acc_b.at[can_v], bb, sem2)
    c1.wait()
    pltpu.sync_copy(bx, xg_h.at[pl.ds(wid * GW, GW)])
    c2.wait()
    pltpu.sync_copy(ba, ga_h.at[pl.ds(wid * GW, GW)])
    c3.wait()
    pltpu.sync_copy(bb, gb_h.at[pl.ds(wid * GW, GW)])


def _vq_body(xg_ref, ga_ref, gb_ref, ws_ref, wn_ref, fw_ref, fb_ref, cb_ref,
             onehot_ref, loss_ref,
             out_s, m_s, z_s, idx_s, div_s):
    p = pl.program_id(0)
    v = pl.program_id(1)

    @pl.when(jnp.logical_and(p == 0, v == 0))
    def _():
        agg = ga_ref[...] + gb_ref[...]
        deg = jnp.maximum(agg[:, D:D + 1], 1.0)
        aggn = agg[:, :D] / deg
        xg = xg_ref[:, :D]
        h = jnp.maximum(
            jnp.dot(xg, ws_ref[...], preferred_element_type=jnp.float32)
            + jnp.dot(aggn, wn_ref[...], preferred_element_type=jnp.float32),
            0.0)
        out = (jnp.dot(h[0:B], fw_ref[0:EMB], preferred_element_type=jnp.float32)
               + jnp.dot(h[1024:1024 + B], fw_ref[EMB:2 * EMB],
                         preferred_element_type=jnp.float32)
               + fb_ref[...])
        out_s[...] = out
        m_s[...] = jnp.full((B, 1), jnp.inf, jnp.float32)
        z_s[...] = jnp.zeros((B, 1), jnp.float32)
        idx_s[...] = jnp.full((B, 1), jnp.int32(2 ** 30))

    out = out_s[...]
    cb = cb_ref[...]
    db = (jnp.sum(out * out, axis=1, keepdims=True)
          - 2.0 * lax.dot_general(out, cb, (((1,), (1,)), ((), ())),
                                  preferred_element_type=jnp.float32)
          + jnp.sum(cb * cb, axis=1)[None, :])
    col = lax.broadcasted_iota(jnp.int32, (B, BV), 1) + v * BV

    @pl.when(p == 0)
    def _():
        bm = jnp.min(db, axis=1, keepdims=True)
        bidx = jnp.min(jnp.where(db == bm, col, jnp.int32(2 ** 30)),
                       axis=1, keepdims=True)
        m_old = m_s[...]
        better = bm < m_old
        m_new = jnp.minimum(m_old, bm)
        idx_s[...] = jnp.where(better, bidx, idx_s[...])
        z_s[...] = (z_s[...] * jnp.exp(m_new - m_old)
                    + jnp.sum(jnp.exp(m_new - db), axis=1, keepdims=True))
        m_s[...] = m_new

    @pl.when(p == 1)
    def _():
        @pl.when(v == 0)
        def _():
            div_s[0, 0] = 0.0
        probs = jnp.exp(m_s[...] - db) / z_s[...]
        avg = jnp.sum(probs, axis=0, keepdims=True) * (1.0 / B)
        div_s[0, 0] += jnp.sum(avg * jnp.log(avg + 1e-9))
        onehot_ref[...] = jnp.where(col == idx_s[...], 1.0, 0.0)

        @pl.when(v == NBV - 1)
        def _():
            commit = 0.2 * jnp.sum(m_s[...]) * (1.0 / (B * HID))
            loss_ref[...] = jnp.full((1, 1), commit + 0.1 * div_s[0, 0],
                                     jnp.float32)


def kernel(x, edge_index, ptr, target_node_idx, ego_node_idx,
           W_self, W_nbr, fc_w, fc_b, codebook):
    f32 = jnp.float32
    i32 = jnp.int32
    xext = jnp.concatenate(
        [x, jnp.ones((N, 1), f32), jnp.zeros((N, XW - D - 1), f32)], axis=1)
    src = edge_index[0].reshape(NW, E // NW)
    dst = edge_index[1].reshape(NW, E // NW)
    padw = EWP - E // NW
    srcp = jnp.concatenate(
        [src, jnp.zeros((NW, padw), i32)], axis=1).reshape(NW, CH, K)
    dstp = jnp.concatenate(
        [dst, jnp.full((NW, padw), N, i32)], axis=1).reshape(NW, CH, K)

    adj_t = target_node_idx + ptr[:-1]
    adj_e = ego_node_idx + ptr[:-1]
    padb = jnp.zeros((1024 - B,), i32)
    idx_g = jnp.concatenate([adj_t, padb, adj_e, padb])
    canon = jnp.argmax(idx_g[None, :] == idx_g[:, None], axis=1).astype(i32)
    slot_tab = jnp.full((NPAD,), DUMMY, i32).at[idx_g].set(canon)
    zero_init = jnp.zeros((ACC_ROWS, XW), f32)
    idx_g2 = idx_g.reshape(NW, GW)
    canon2 = canon.reshape(NW, GW)

    acc_a, acc_b = _edge_agg(xext, srcp, dstp, slot_tab, zero_init)
    xg, ga, gb = _row_gather(xext, acc_a, acc_b, idx_g2, canon2)

    onehot, loss = pl.pallas_call(
        _vq_body,
        grid=(2, NBV),
        in_specs=[
            pl.BlockSpec((G, XW), lambda p, v: (0, 0)),
            pl.BlockSpec((G, XW), lambda p, v: (0, 0)),
            pl.BlockSpec((G, XW), lambda p, v: (0, 0)),
            pl.BlockSpec((D, EMB), lambda p, v: (0, 0)),
            pl.BlockSpec((D, EMB), lambda p, v: (0, 0)),
            pl.BlockSpec((2 * EMB, HID), lambda p, v: (0, 0)),
            pl.BlockSpec((1, HID), lambda p, v: (0, 0)),
            pl.BlockSpec((BV, HID), lambda p, v: (v, 0)),
        ],
        out_specs=[
            pl.BlockSpec((B, BV), lambda p, v: (0, v)),
            pl.BlockSpec((1, 1), lambda p, v: (0, 0)),
        ],
        out_shape=[
            jax.ShapeDtypeStruct((B, VOCAB), f32),
            jax.ShapeDtypeStruct((1, 1), f32),
        ],
        scratch_shapes=[
            pltpu.VMEM((B, HID), f32),
            pltpu.VMEM((B, 1), f32),
            pltpu.VMEM((B, 1), f32),
            pltpu.VMEM((B, 1), jnp.int32),
            pltpu.SMEM((1, 1), f32),
        ],
    )(xg, ga, gb, W_self, W_nbr, fc_w, fc_b.reshape(1, HID), codebook)
    return onehot, loss[0, 0]


# speedup vs baseline: 4.9949x; 1.0856x over previous
import functools

import jax
import jax.numpy as jnp
from jax import lax
from jax.experimental import pallas as pl
from jax.experimental.pallas import tpu as pltpu
from jax.experimental.pallas import tpu_sc as plsc

N = 10000
E = 320000
D = 128
EMB = 256
HID = 512
VOCAB = 8192
B = 1000

NW = 32
K = 128
CH = 80
EWP = CH * K
XW = 144
NPAD = N + 112
G = 2048
GW = G // NW
DUMMY = G
ACC_ROWS = 2176
RPT = ACC_ROWS // 16
NBV = 8
BV = VOCAB // NBV


def _sc_mesh():
    return plsc.VectorSubcoreMesh(core_axis_name="c", subcore_axis_name="s")


_SC_PARAMS = pltpu.CompilerParams(use_tc_tiling_on_sc=False,
                                  needs_layout_passes=False)


@functools.partial(
    pl.kernel,
    mesh=_sc_mesh(),
    compiler_params=_SC_PARAMS,
    out_type=(
        jax.ShapeDtypeStruct((ACC_ROWS, XW), jnp.float32),
        jax.ShapeDtypeStruct((ACC_ROWS, XW), jnp.float32),
    ),
    scratch_types=[
        pltpu.VMEM((CH, K), jnp.int32),
        pltpu.VMEM((CH, K), jnp.int32),
        pltpu.VMEM((NPAD,), jnp.int32),
        pltpu.VMEM((K, XW), jnp.float32),
        pltpu.VMEM((K, XW), jnp.float32),
        pltpu.VMEM((K, XW), jnp.float32),
        pltpu.VMEM((K, XW), jnp.float32),
        pltpu.VMEM_SHARED((ACC_ROWS, XW), jnp.float32),
        pltpu.SemaphoreType.DMA,
        pltpu.SemaphoreType.DMA,
        pltpu.SemaphoreType.DMA,
        pltpu.SemaphoreType.DMA,
    ],
)
def _edge_agg(xext_h, srcp_h, dstp_h, slot_h, zero_h, out_a, out_b,
              src_v, dst_v, slot_v, r0, r1, r2, r3, acc_sh,
              sem0, sem1, sem2, sem3):
    cid = lax.axis_index("c")
    sid = lax.axis_index("s")
    wid = sid * 2 + cid
    pltpu.sync_copy(zero_h.at[pl.ds(sid * RPT, RPT)],
                    acc_sh.at[pl.ds(sid * RPT, RPT)])
    plsc.subcore_barrier()
    pltpu.sync_copy(srcp_h.at[wid], src_v)
    pltpu.sync_copy(dstp_h.at[wid], dst_v)
    pltpu.sync_copy(slot_h, slot_v)

    def translate(j, carry):
        for k in range(K // 16):
            d16 = dst_v[j, pl.ds(k * 16, 16)]
            dst_v[j, pl.ds(k * 16, 16)] = plsc.load_gather(slot_v, [d16])
        return carry

    lax.fori_loop(0, CH, translate, 0)

    bufs = (r0, r1, r2, r3)
    sems = (sem0, sem1, sem2, sem3)
    NB = len(bufs)

    def group(g, carry):
        base = g * NB
        copies = [
            pltpu.async_copy(xext_h.at[src_v.at[base + i]], bufs[i], sems[i])
            for i in range(NB)
        ]
        for i in range(NB):
            copies[i].wait()
            pltpu.sync_copy(bufs[i], acc_sh.at[dst_v.at[base + i]], add=True)
        return carry

    lax.fori_loop(0, CH // NB, group, 0)
    plsc.subcore_barrier()

    @pl.when(cid == 0)
    def _():
        pltpu.sync_copy(acc_sh.at[pl.ds(sid * RPT, RPT)],
                        out_a.at[pl.ds(sid * RPT, RPT)])

    @pl.when(cid == 1)
    def _():
        pltpu.sync_copy(acc_sh.at[pl.ds(sid * RPT, RPT)],
                        out_b.at[pl.ds(sid * RPT, RPT)])


@functools.partial(
    pl.kernel,
    mesh=_sc_mesh(),
    compiler_params=_SC_PARAMS,
    out_type=(
        jax.ShapeDtypeStruct((G, XW), jnp.float32),
        jax.ShapeDtypeStruct((G, XW), jnp.float32),
        jax.ShapeDtypeStruct((G, XW), jnp.float32),
    ),
    scratch_types=[
        pltpu.VMEM((GW,), jnp.int32),
        pltpu.VMEM((GW,), jnp.int32),
        pltpu.VMEM((GW, XW), jnp.float32),
        pltpu.VMEM((GW, XW), jnp.float32),
        pltpu.VMEM((GW, XW), jnp.float32),
        pltpu.SemaphoreType.DMA,
        pltpu.SemaphoreType.DMA,
        pltpu.SemaphoreType.DMA,
    ],
)
def _row_gather(xext_h, acc_a, acc_b, idx_h, canon_h, xg_h, ga_h, gb_h,
                idx_v, can_v, bx, ba, bb, sem0, sem1, sem2):
    cid = lax.axis_index("c")
    sid = lax.axis_index("s")
    wid = sid * 2 + cid
    pltpu.sync_copy(idx_h.at[wid], idx_v)
    pltpu.sync_copy(canon_h.at[wid], can_v)
    c1 = pltpu.async_copy(xext_h.at[idx_v], bx, sem0)
    c2 = pltpu.async_copy(acc_a.at[can_v], ba, sem1)
    c3 = pltpu.async_copy(acc_b.at[can_v], bb, sem2)
    c1.wait()
    pltpu.sync_copy(bx, xg_h.at[pl.ds(wid * GW, GW)])
    c2.wait()
    pltpu.sync_copy(ba, ga_h.at[pl.ds(wid * GW, GW)])
    c3.wait()
    pltpu.sync_copy(bb, gb_h.at[pl.ds(wid * GW, GW)])


def _vq_body(xg_ref, ga_ref, gb_ref, ws_ref, wn_ref, fw_ref, fb_ref, cb_ref,
             onehot_ref, loss_ref,
             out_s, m_s, z_s, idx_s, div_s):
    p = pl.program_id(0)
    v = pl.program_id(1)

    @pl.when(jnp.logical_and(p == 0, v == 0))
    def _():
        agg = ga_ref[...] + gb_ref[...]
        deg = jnp.maximum(agg[:, D:D + 1], 1.0)
        aggn = agg[:, :D] / deg
        xg = xg_ref[:, :D]
        h = jnp.maximum(
            jnp.dot(xg, ws_ref[...], preferred_element_type=jnp.float32)
            + jnp.dot(aggn, wn_ref[...], preferred_element_type=jnp.float32),
            0.0)
        out = (jnp.dot(h[0:B], fw_ref[0:EMB], preferred_element_type=jnp.float32)
               + jnp.dot(h[1024:1024 + B], fw_ref[EMB:2 * EMB],
                         preferred_element_type=jnp.float32)
               + fb_ref[...])
        out_s[...] = out
        m_s[...] = jnp.full((B, 1), jnp.inf, jnp.float32)
        z_s[...] = jnp.zeros((B, 1), jnp.float32)
        idx_s[...] = jnp.full((B, 1), jnp.int32(2 ** 30))

    out = out_s[...]
    cb = cb_ref[...]
    db = (jnp.sum(out * out, axis=1, keepdims=True)
          - 2.0 * lax.dot_general(out, cb, (((1,), (1,)), ((), ())),
                                  preferred_element_type=jnp.float32)
          + jnp.sum(cb * cb, axis=1)[None, :])
    col = lax.broadcasted_iota(jnp.int32, (B, BV), 1) + v * BV

    @pl.when(p == 0)
    def _():
        bm = jnp.min(db, axis=1, keepdims=True)
        bidx = jnp.min(jnp.where(db == bm, col, jnp.int32(2 ** 30)),
                       axis=1, keepdims=True)
        m_old = m_s[...]
        better = bm < m_old
        m_new = jnp.minimum(m_old, bm)
        idx_s[...] = jnp.where(better, bidx, idx_s[...])
        z_s[...] = (z_s[...] * jnp.exp(m_new - m_old)
                    + jnp.sum(jnp.exp(m_new - db), axis=1, keepdims=True))
        m_s[...] = m_new

    @pl.when(p == 1)
    def _():
        @pl.when(v == 0)
        def _():
            div_s[0, 0] = 0.0
        probs = jnp.exp(m_s[...] - db) / z_s[...]
        avg = jnp.sum(probs, axis=0, keepdims=True) * (1.0 / B)
        div_s[0, 0] += jnp.sum(avg * jnp.log(avg + 1e-9))
        onehot_ref[...] = jnp.where(col == idx_s[...], 1.0, 0.0)

        @pl.when(v == NBV - 1)
        def _():
            commit = 0.2 * jnp.sum(m_s[...]) * (1.0 / (B * HID))
            loss_ref[...] = jnp.full((1, 1), commit + 0.1 * div_s[0, 0],
                                     jnp.float32)


def kernel(x, edge_index, ptr, target_node_idx, ego_node_idx,
           W_self, W_nbr, fc_w, fc_b, codebook):
    f32 = jnp.float32
    i32 = jnp.int32
    xext = jnp.concatenate(
        [x, jnp.ones((N, 1), f32), jnp.zeros((N, XW - D - 1), f32)], axis=1)
    src = edge_index[0].reshape(NW, E // NW)
    dst = edge_index[1].reshape(NW, E // NW)
    padw = EWP - E // NW
    srcp = jnp.concatenate(
        [src, jnp.zeros((NW, padw), i32)], axis=1).reshape(NW, CH, K)
    dstp = jnp.concatenate(
        [dst, jnp.full((NW, padw), N, i32)], axis=1).reshape(NW, CH, K)

    adj_t = target_node_idx + ptr[:-1]
    adj_e = ego_node_idx + ptr[:-1]
    padb = jnp.zeros((1024 - B,), i32)
    idx_g = jnp.concatenate([adj_t, padb, adj_e, padb])
    canon = jnp.argmax(idx_g[None, :] == idx_g[:, None], axis=1).astype(i32)
    dummy_spread = DUMMY + (jnp.arange(NPAD, dtype=i32) % (ACC_ROWS - G))
    slot_tab = dummy_spread.at[idx_g].set(canon)
    zero_init = jnp.zeros((ACC_ROWS, XW), f32)
    idx_g2 = idx_g.reshape(NW, GW)
    canon2 = canon.reshape(NW, GW)

    acc_a, acc_b = _edge_agg(xext, srcp, dstp, slot_tab, zero_init)
    xg, ga, gb = _row_gather(xext, acc_a, acc_b, idx_g2, canon2)

    onehot, loss = pl.pallas_call(
        _vq_body,
        grid=(2, NBV),
        in_specs=[
            pl.BlockSpec((G, XW), lambda p, v: (0, 0)),
            pl.BlockSpec((G, XW), lambda p, v: (0, 0)),
            pl.BlockSpec((G, XW), lambda p, v: (0, 0)),
            pl.BlockSpec((D, EMB), lambda p, v: (0, 0)),
            pl.BlockSpec((D, EMB), lambda p, v: (0, 0)),
            pl.BlockSpec((2 * EMB, HID), lambda p, v: (0, 0)),
            pl.BlockSpec((1, HID), lambda p, v: (0, 0)),
            pl.BlockSpec((BV, HID), lambda p, v: (v, 0)),
        ],
        out_specs=[
            pl.BlockSpec((B, BV), lambda p, v: (0, v)),
            pl.BlockSpec((1, 1), lambda p, v: (0, 0)),
        ],
        out_shape=[
            jax.ShapeDtypeStruct((B, VOCAB), f32),
            jax.ShapeDtypeStruct((1, 1), f32),
        ],
        scratch_shapes=[
            pltpu.VMEM((B, HID), f32),
            pltpu.VMEM((B, 1), f32),
            pltpu.VMEM((B, 1), f32),
            pltpu.VMEM((B, 1), jnp.int32),
            pltpu.SMEM((1, 1), f32),
        ],
    )(xg, ga, gb, W_self, W_nbr, fc_w, fc_b.reshape(1, HID), codebook)
    return onehot, loss[0, 0]


# speedup vs baseline: 8.1426x; 1.6302x over previous
import functools

import jax
import jax.numpy as jnp
from jax import lax
from jax.experimental import pallas as pl
from jax.experimental.pallas import tpu as pltpu
from jax.experimental.pallas import tpu_sc as plsc

N = 10000
E = 320000
D = 128
EMB = 256
HID = 512
VOCAB = 8192
B = 1000

NW = 32
K = 128
CH = 80
EWP = CH * K
XW = 144
NPAD = N + 112
G = 2048
GW = G // NW
DUMMY = G
ACC_ROWS = 2176
RPT = ACC_ROWS // 16
NBV = 8
BV = VOCAB // NBV


def _sc_mesh():
    return plsc.VectorSubcoreMesh(core_axis_name="c", subcore_axis_name="s")


_SC_PARAMS = pltpu.CompilerParams(use_tc_tiling_on_sc=False,
                                  needs_layout_passes=False)


@functools.partial(
    pl.kernel,
    mesh=_sc_mesh(),
    compiler_params=_SC_PARAMS,
    out_type=(
        jax.ShapeDtypeStruct((ACC_ROWS, XW), jnp.float32),
        jax.ShapeDtypeStruct((ACC_ROWS, XW), jnp.float32),
    ),
    scratch_types=[
        pltpu.VMEM((EWP,), jnp.int32),
        pltpu.VMEM((EWP,), jnp.int32),
        pltpu.VMEM((EWP + 384,), jnp.int32),
        pltpu.VMEM((EWP + 384,), jnp.int32),
        pltpu.VMEM((NPAD,), jnp.int32),
        pltpu.VMEM((K, XW), jnp.float32),
        pltpu.VMEM((K, XW), jnp.float32),
        pltpu.VMEM((K, XW), jnp.float32),
        pltpu.VMEM_SHARED((ACC_ROWS, XW), jnp.float32),
        pltpu.SemaphoreType.DMA,
        pltpu.SemaphoreType.DMA,
        pltpu.SemaphoreType.DMA,
    ],
)
def _edge_agg(xext_h, srcp_h, dstp_h, slot_h, zero_h, out_a, out_b,
              src_v, dst_v, csrc_v, cslot_v, slot_v, r0, r1, r2, acc_sh,
              sem0, sem1, sem2):
    cid = lax.axis_index("c")
    sid = lax.axis_index("s")
    wid = sid * 2 + cid
    pltpu.sync_copy(zero_h.at[pl.ds(sid * RPT, RPT)],
                    acc_sh.at[pl.ds(sid * RPT, RPT)])
    plsc.subcore_barrier()
    pltpu.sync_copy(srcp_h.at[wid], src_v)
    pltpu.sync_copy(dstp_h.at[wid], dst_v)
    pltpu.sync_copy(slot_h, slot_v)

    def compact(i, cnt):
        s16 = src_v[pl.ds(i * 16, 16)]
        d16 = dst_v[pl.ds(i * 16, 16)]
        sl16 = plsc.load_gather(slot_v, [d16])
        valid = sl16 < DUMMY
        plsc.store_compressed(csrc_v.at[pl.ds(cnt, 16)], s16, mask=valid)
        plsc.store_compressed(cslot_v.at[pl.ds(cnt, 16)], sl16, mask=valid)
        return cnt + jnp.max(plsc.all_reduce_population_count(valid))

    cnt = lax.fori_loop(0, EWP // 16, compact, jnp.int32(0))

    zeros16 = jnp.zeros((16,), jnp.int32)
    dummy16 = jnp.full((16,), DUMMY, jnp.int32)
    for k in range(3 * K // 16):
        csrc_v[pl.ds(cnt + k * 16, 16)] = zeros16
        cslot_v[pl.ds(cnt + k * 16, 16)] = dummy16

    bufs = (r0, r1, r2)
    sems = (sem0, sem1, sem2)
    NB = len(bufs)
    ngroups = (cnt + NB * K - 1) // (NB * K)

    def group(g, carry):
        base = g * NB * K
        copies = [
            pltpu.async_copy(
                xext_h.at[csrc_v.at[pl.ds(base + i * K, K)]], bufs[i], sems[i])
            for i in range(NB)
        ]
        for i in range(NB):
            copies[i].wait()
            pltpu.sync_copy(bufs[i],
                            acc_sh.at[cslot_v.at[pl.ds(base + i * K, K)]],
                            add=True)
        return carry

    lax.fori_loop(0, ngroups, group, 0)
    plsc.subcore_barrier()

    @pl.when(cid == 0)
    def _():
        pltpu.sync_copy(acc_sh.at[pl.ds(sid * RPT, RPT)],
                        out_a.at[pl.ds(sid * RPT, RPT)])

    @pl.when(cid == 1)
    def _():
        pltpu.sync_copy(acc_sh.at[pl.ds(sid * RPT, RPT)],
                        out_b.at[pl.ds(sid * RPT, RPT)])


@functools.partial(
    pl.kernel,
    mesh=_sc_mesh(),
    compiler_params=_SC_PARAMS,
    out_type=(
        jax.ShapeDtypeStruct((G, XW), jnp.float32),
        jax.ShapeDtypeStruct((G, XW), jnp.float32),
        jax.ShapeDtypeStruct((G, XW), jnp.float32),
    ),
    scratch_types=[
        pltpu.VMEM((GW,), jnp.int32),
        pltpu.VMEM((GW,), jnp.int32),
        pltpu.VMEM((GW, XW), jnp.float32),
        pltpu.VMEM((GW, XW), jnp.float32),
        pltpu.VMEM((GW, XW), jnp.float32),
        pltpu.SemaphoreType.DMA,
        pltpu.SemaphoreType.DMA,
        pltpu.SemaphoreType.DMA,
    ],
)
def _row_gather(xext_h, acc_a, acc_b, idx_h, canon_h, xg_h, ga_h, gb_h,
                idx_v, can_v, bx, ba, bb, sem0, sem1, sem2):
    cid = lax.axis_index("c")
    sid = lax.axis_index("s")
    wid = sid * 2 + cid
    pltpu.sync_copy(idx_h.at[wid], idx_v)
    pltpu.sync_copy(canon_h.at[wid], can_v)
    c1 = pltpu.async_copy(xext_h.at[idx_v], bx, sem0)
    c2 = pltpu.async_copy(acc_a.at[can_v], ba, sem1)
    c3 = pltpu.async_copy(acc_b.at[can_v], bb, sem2)
    c1.wait()
    pltpu.sync_copy(bx, xg_h.at[pl.ds(wid * GW, GW)])
    c2.wait()
    pltpu.sync_copy(ba, ga_h.at[pl.ds(wid * GW, GW)])
    c3.wait()
    pltpu.sync_copy(bb, gb_h.at[pl.ds(wid * GW, GW)])


def _vq_body(xg_ref, ga_ref, gb_ref, ws_ref, wn_ref, fw_ref, fb_ref, cb_ref,
             onehot_ref, loss_ref,
             out_s, m_s, z_s, idx_s, div_s):
    p = pl.program_id(0)
    v = pl.program_id(1)

    @pl.when(jnp.logical_and(p == 0, v == 0))
    def _():
        agg = ga_ref[...] + gb_ref[...]
        deg = jnp.maximum(agg[:, D:D + 1], 1.0)
        aggn = agg[:, :D] / deg
        xg = xg_ref[:, :D]
        h = jnp.maximum(
            jnp.dot(xg, ws_ref[...], preferred_element_type=jnp.float32)
            + jnp.dot(aggn, wn_ref[...], preferred_element_type=jnp.float32),
            0.0)
        out = (jnp.dot(h[0:B], fw_ref[0:EMB], preferred_element_type=jnp.float32)
               + jnp.dot(h[1024:1024 + B], fw_ref[EMB:2 * EMB],
                         preferred_element_type=jnp.float32)
               + fb_ref[...])
        out_s[...] = out
        m_s[...] = jnp.full((B, 1), jnp.inf, jnp.float32)
        z_s[...] = jnp.zeros((B, 1), jnp.float32)
        idx_s[...] = jnp.full((B, 1), jnp.int32(2 ** 30))

    out = out_s[...]
    cb = cb_ref[...]
    db = (jnp.sum(out * out, axis=1, keepdims=True)
          - 2.0 * lax.dot_general(out, cb, (((1,), (1,)), ((), ())),
                                  preferred_element_type=jnp.float32)
          + jnp.sum(cb * cb, axis=1)[None, :])
    col = lax.broadcasted_iota(jnp.int32, (B, BV), 1) + v * BV

    @pl.when(p == 0)
    def _():
        bm = jnp.min(db, axis=1, keepdims=True)
        bidx = jnp.min(jnp.where(db == bm, col, jnp.int32(2 ** 30)),
                       axis=1, keepdims=True)
        m_old = m_s[...]
        better = bm < m_old
        m_new = jnp.minimum(m_old, bm)
        idx_s[...] = jnp.where(better, bidx, idx_s[...])
        z_s[...] = (z_s[...] * jnp.exp(m_new - m_old)
                    + jnp.sum(jnp.exp(m_new - db), axis=1, keepdims=True))
        m_s[...] = m_new

    @pl.when(p == 1)
    def _():
        @pl.when(v == 0)
        def _():
            div_s[0, 0] = 0.0
        probs = jnp.exp(m_s[...] - db) / z_s[...]
        avg = jnp.sum(probs, axis=0, keepdims=True) * (1.0 / B)
        div_s[0, 0] += jnp.sum(avg * jnp.log(avg + 1e-9))
        onehot_ref[...] = jnp.where(col == idx_s[...], 1.0, 0.0)

        @pl.when(v == NBV - 1)
        def _():
            commit = 0.2 * jnp.sum(m_s[...]) * (1.0 / (B * HID))
            loss_ref[...] = jnp.full((1, 1), commit + 0.1 * div_s[0, 0],
                                     jnp.float32)


def kernel(x, edge_index, ptr, target_node_idx, ego_node_idx,
           W_self, W_nbr, fc_w, fc_b, codebook):
    f32 = jnp.float32
    i32 = jnp.int32
    xext = jnp.concatenate(
        [x, jnp.ones((N, 1), f32), jnp.zeros((N, XW - D - 1), f32)], axis=1)
    src = edge_index[0].reshape(NW, E // NW)
    dst = edge_index[1].reshape(NW, E // NW)
    padw = EWP - E // NW
    srcp = jnp.concatenate([src, jnp.zeros((NW, padw), i32)], axis=1)
    dstp = jnp.concatenate([dst, jnp.full((NW, padw), N, i32)], axis=1)

    adj_t = target_node_idx + ptr[:-1]
    adj_e = ego_node_idx + ptr[:-1]
    padb = jnp.zeros((1024 - B,), i32)
    idx_g = jnp.concatenate([adj_t, padb, adj_e, padb])
    canon = jnp.argmax(idx_g[None, :] == idx_g[:, None], axis=1).astype(i32)
    dummy_spread = DUMMY + (jnp.arange(NPAD, dtype=i32) % (ACC_ROWS - G))
    slot_tab = dummy_spread.at[idx_g].set(canon)
    zero_init = jnp.zeros((ACC_ROWS, XW), f32)
    idx_g2 = idx_g.reshape(NW, GW)
    canon2 = canon.reshape(NW, GW)

    acc_a, acc_b = _edge_agg(xext, srcp, dstp, slot_tab, zero_init)
    xg, ga, gb = _row_gather(xext, acc_a, acc_b, idx_g2, canon2)

    onehot, loss = pl.pallas_call(
        _vq_body,
        grid=(2, NBV),
        in_specs=[
            pl.BlockSpec((G, XW), lambda p, v: (0, 0)),
            pl.BlockSpec((G, XW), lambda p, v: (0, 0)),
            pl.BlockSpec((G, XW), lambda p, v: (0, 0)),
            pl.BlockSpec((D, EMB), lambda p, v: (0, 0)),
            pl.BlockSpec((D, EMB), lambda p, v: (0, 0)),
            pl.BlockSpec((2 * EMB, HID), lambda p, v: (0, 0)),
            pl.BlockSpec((1, HID), lambda p, v: (0, 0)),
            pl.BlockSpec((BV, HID), lambda p, v: (v, 0)),
        ],
        out_specs=[
            pl.BlockSpec((B, BV), lambda p, v: (0, v)),
            pl.BlockSpec((1, 1), lambda p, v: (0, 0)),
        ],
        out_shape=[
            jax.ShapeDtypeStruct((B, VOCAB), f32),
            jax.ShapeDtypeStruct((1, 1), f32),
        ],
        scratch_shapes=[
            pltpu.VMEM((B, HID), f32),
            pltpu.VMEM((B, 1), f32),
            pltpu.VMEM((B, 1), f32),
            pltpu.VMEM((B, 1), jnp.int32),
            pltpu.SMEM((1, 1), f32),
        ],
    )(xg, ga, gb, W_self, W_nbr, fc_w, fc_b.reshape(1, HID), codebook)
    return onehot, loss[0, 0]
